# baseline (speedup 1.0000x reference)
import functools

import jax
import jax.numpy as jnp
from jax import lax
from jax.experimental import pallas as pl
from jax.experimental.pallas import tpu as pltpu
from jax.experimental.pallas import tpu_sc as plsc

_LANES = 16
_NW = 32


def _sc_interp(idx, wl_b, wr_b, g0, g1, B, H, W, F):
  chunk = W * F
  npairs = 2 * B
  mesh = plsc.VectorSubcoreMesh(core_axis_name="c", subcore_axis_name="s")

  @functools.partial(
      pl.kernel,
      mesh=mesh,
      out_type=jax.ShapeDtypeStruct((npairs, B, H * W * F), jnp.float32),
      scratch_types=[
          pltpu.VMEM((npairs,), jnp.int32),
          pltpu.VMEM((B, _LANES), jnp.float32),
          pltpu.VMEM((B, _LANES), jnp.float32),
          pltpu.VMEM((npairs, chunk), jnp.float32),
          pltpu.VMEM((npairs, chunk), jnp.float32),
          pltpu.VMEM((2, B, chunk), jnp.float32),
          pltpu.SemaphoreType.DMA,
          pltpu.SemaphoreType.DMA,
          pltpu.SemaphoreType.DMA,
      ],
  )
  def sck(idx_hbm, wl_hbm, wr_hbm, g0_hbm, g1_hbm, out_hbm,
          idx_v, wl_v, wr_v, rows0, rows1, ob, gsem, osem0, osem1):
    h = lax.axis_index("s") * 2 + lax.axis_index("c")
    pltpu.sync_copy(idx_hbm.at[h], idx_v)
    pltpu.sync_copy(wl_hbm, wl_v)
    pltpu.sync_copy(wr_hbm, wr_v)
    cp0 = pltpu.async_copy(g0_hbm.at[idx_v], rows0, gsem)
    cp1 = pltpu.async_copy(g1_hbm.at[idx_v], rows1, gsem)
    cp0.wait()
    cp1.wait()

    wls = [wl_v[b2] for b2 in range(B)]
    wrs = [wr_v[b2] for b2 in range(B)]
    osems = [osem0, osem1]
    pending = [None, None]

    for p in range(npairs):
      b1, c = p // 2, p % 2
      rows = rows0 if c == 0 else rows1
      buf = p % 2
      if pending[buf] is not None:
        pending[buf].wait()

      def body(j, _, rows=rows, buf=buf, b1=b1):
        off = j * _LANES
        gl = rows[2 * b1, pl.ds(off, _LANES)]
        gr = rows[2 * b1 + 1, pl.ds(off, _LANES)]
        for b2 in range(B):
          ob[buf, b2, pl.ds(off, _LANES)] = wrs[b2] * gl + wls[b2] * gr
        return 0

      lax.fori_loop(0, chunk // _LANES, body, 0)
      pending[buf] = pltpu.async_copy(
          ob.at[buf], out_hbm.at[p, :, pl.ds(h * chunk, chunk)], osems[buf])

    for cp in pending:
      if cp is not None:
        cp.wait()

  return sck(idx, wl_b, wr_b, g0, g1)


def kernel(patch_indices, grid0, grid1):
  T, F, H, W = grid0.shape
  B = patch_indices.shape[0]

  t = patch_indices[:, 0, 0, 0] * T
  left = jnp.floor(t).astype(jnp.int32)
  right = jnp.clip(left + 1, 0, T - 1)
  dl = t - left.astype(t.dtype)
  dr = right.astype(t.dtype) - t

  lr = jnp.stack([left, right], axis=1).reshape(-1)
  h_ids = jnp.arange(H, dtype=jnp.int32)[:, None]
  idx = lr[None, :] * H + h_ids
  wl_b = jnp.broadcast_to(dl[:, None], (B, _LANES))
  wr_b = jnp.broadcast_to(dr[:, None], (B, _LANES))

  g0 = grid0.transpose(0, 2, 3, 1).reshape(T * H, W * F)
  g1 = grid1.transpose(0, 2, 3, 1).reshape(T * H, W * F)
  out = _sc_interp(idx, wl_b, wr_b, g0, g1, B, H, W, F)
  return out.reshape(B, 2, B, H, W, F).transpose(0, 1, 2, 5, 3, 4)


# speedup vs baseline: 1.8902x; 1.8902x over previous
import functools

import jax
import jax.numpy as jnp
from jax import lax
from jax.experimental import pallas as pl
from jax.experimental.pallas import tpu as pltpu
from jax.experimental.pallas import tpu_sc as plsc

_LANES = 16
_NW = 32
_FC = 16
_WC = 8
_TP = 256


def _sc_interp(lr_b, wl_b, wr_b, gt0, gt1, B, T, F, H, W):
  chunk = W * F
  npairs = 2 * B
  nfc = F // _FC
  nwc = W // _WC
  nj = chunk // _LANES
  mesh = plsc.VectorSubcoreMesh(core_axis_name="c", subcore_axis_name="s")

  @functools.partial(
      pl.kernel,
      mesh=mesh,
      compiler_params=pltpu.CompilerParams(needs_layout_passes=False),
      out_type=jax.ShapeDtypeStruct((npairs, B, H * chunk), jnp.float32),
      scratch_types=[
          pltpu.VMEM((npairs, _LANES), jnp.int32),
          pltpu.VMEM((B, _LANES), jnp.float32),
          pltpu.VMEM((B, _LANES), jnp.float32),
          pltpu.VMEM((_FC, _WC, T), jnp.float32),
          pltpu.VMEM((_FC, _WC, T), jnp.float32),
          pltpu.VMEM((npairs, chunk), jnp.float32),
          pltpu.VMEM((B, chunk), jnp.float32),
          pltpu.SemaphoreType.DMA,
          pltpu.SemaphoreType.DMA,
          pltpu.SemaphoreType.DMA,
      ],
  )
  def sck(lr_hbm, wl_hbm, wr_hbm, g0_hbm, g1_hbm, out_hbm,
          lr_v, wl_v, wr_v, wbuf0, wbuf1, exb, ob, wsem0, wsem1, osem):
    h = lax.axis_index("s") * 2 + lax.axis_index("c")
    pltpu.sync_copy(lr_hbm, lr_v)
    pltpu.sync_copy(wl_hbm, wl_v)
    pltpu.sync_copy(wr_hbm, wr_v)

    wsems = [wsem0, wsem1]
    wbufs = [wbuf0, wbuf1]
    pend_w = [None, None]
    pend_o = [None]
    nwin = 2 * nfc * nwc

    def issue_w(win):
      c, fw = win // (nfc * nwc), win % (nfc * nwc)
      fc, wc = fw // nwc, fw % nwc
      g = g0_hbm if c == 0 else g1_hbm
      wb = win % 2
      pend_w[wb] = pltpu.async_copy(
          g.at[pl.ds(fc * _FC, _FC), h, pl.ds(wc * _WC, _WC), :],
          wbufs[wb], wsems[wb])

    issue_w(0)
    issue_w(1)

    for c in range(2):
      for fw in range(nfc * nwc):
        fc, wc = fw // nwc, fw % nwc
        win = c * nfc * nwc + fw
        wb = win % 2
        pend_w[wb].wait()
        def wbody(wli, _, wb=wb, wc=wc, fc=fc):
          off = (wc * _WC + wli) * F + fc * _FC
          iot = lax.iota(jnp.int32, _LANES)
          wliv = jnp.full((_LANES,), wli, dtype=jnp.int32)
          for k in range(npairs):
            vals = plsc.load_gather(wbufs[wb], [iot, wliv, lr_v[k]])
            exb[k, pl.ds(off, _LANES)] = vals
          return 0

        lax.fori_loop(0, _WC, wbody, 0)
        if win + 2 < nwin:
          issue_w(win + 2)

      for b1 in range(B):
        p = b1 * 2 + c
        if pend_o[0] is not None:
          pend_o[0].wait()

        def cbody(j, _, b1=b1):
          off = j * _LANES
          gl = exb[2 * b1, pl.ds(off, _LANES)]
          gr = exb[2 * b1 + 1, pl.ds(off, _LANES)]
          for b2 in range(B):
            ob[b2, pl.ds(off, _LANES)] = wr_v[b2] * gl + wl_v[b2] * gr
          return 0

        lax.fori_loop(0, nj, cbody, 0)
        pend_o[0] = pltpu.async_copy(
            ob, out_hbm.at[p, :, pl.ds(h * chunk, chunk)], osem)

    if pend_o[0] is not None:
      pend_o[0].wait()

  return sck(lr_b, wl_b, wr_b, gt0, gt1)


def kernel(patch_indices, grid0, grid1):
  T, F, H, W = grid0.shape
  B = patch_indices.shape[0]

  t = patch_indices[:, 0, 0, 0] * T
  left = jnp.floor(t).astype(jnp.int32)
  right = jnp.clip(left + 1, 0, T - 1)
  dl = t - left.astype(t.dtype)
  dr = right.astype(t.dtype) - t

  lr = jnp.stack([left, right], axis=1).reshape(-1)
  lr_b = jnp.broadcast_to(lr[:, None], (2 * B, _LANES))
  wl_b = jnp.broadcast_to(dl[:, None], (B, _LANES))
  wr_b = jnp.broadcast_to(dr[:, None], (B, _LANES))

  gt0 = grid0.transpose(1, 2, 3, 0)
  gt1 = grid1.transpose(1, 2, 3, 0)
  out = _sc_interp(lr_b, wl_b, wr_b, gt0, gt1, B, T, F, H, W)
  return out.reshape(B, 2, B, H, W, F).transpose(0, 1, 2, 5, 3, 4)


# speedup vs baseline: 2.1919x; 1.1596x over previous
import functools

import jax
import jax.numpy as jnp
from jax import lax
from jax.experimental import pallas as pl
from jax.experimental.pallas import tpu as pltpu
from jax.experimental.pallas import tpu_sc as plsc

_LANES = 16
_NW = 32
_FC = 16
_WC = 8
_TP = 256


def _sc_interp(lr_b, wl_b, wr_b, gt0, gt1, B, T, F, H, W):
  chunk = W * F
  npairs = 2 * B
  nfc = F // _FC
  nwc = W // _WC
  nj = chunk // _LANES
  mesh = plsc.VectorSubcoreMesh(core_axis_name="c", subcore_axis_name="s")

  @functools.partial(
      pl.kernel,
      mesh=mesh,
      compiler_params=pltpu.CompilerParams(needs_layout_passes=False),
      out_type=jax.ShapeDtypeStruct((npairs, B, H * chunk), jnp.float32),
      scratch_types=[
          pltpu.VMEM((1, _LANES), jnp.int32),
          pltpu.VMEM((B, _LANES), jnp.float32),
          pltpu.VMEM((B, _LANES), jnp.float32),
          pltpu.VMEM((_FC, _WC, T), jnp.float32),
          pltpu.VMEM((_FC, _WC, T), jnp.float32),
          pltpu.VMEM((npairs * (chunk + 1),), jnp.float32),
          pltpu.VMEM((B, chunk), jnp.float32),
          pltpu.SemaphoreType.DMA,
          pltpu.SemaphoreType.DMA,
          pltpu.SemaphoreType.DMA,
      ],
  )
  def sck(lr_hbm, wl_hbm, wr_hbm, g0_hbm, g1_hbm, out_hbm,
          lr_v, wl_v, wr_v, wbuf0, wbuf1, exb, ob, wsem0, wsem1, osem):
    h = lax.axis_index("s") * 2 + lax.axis_index("c")
    pltpu.sync_copy(lr_hbm, lr_v)
    pltpu.sync_copy(wl_hbm, wl_v)
    pltpu.sync_copy(wr_hbm, wr_v)

    wsems = [wsem0, wsem1]
    wbufs = [wbuf0, wbuf1]
    pend_w = [None, None]
    pend_o = [None]
    nwin = 2 * nfc * nwc

    def issue_w(win):
      c, fw = win // (nfc * nwc), win % (nfc * nwc)
      fc, wc = fw // nwc, fw % nwc
      g = g0_hbm if c == 0 else g1_hbm
      wb = win % 2
      pend_w[wb] = pltpu.async_copy(
          g.at[pl.ds(fc * _FC, _FC), h, pl.ds(wc * _WC, _WC), :],
          wbufs[wb], wsems[wb])

    issue_w(0)
    issue_w(1)

    for c in range(2):
      for fw in range(nfc * nwc):
        fc, wc = fw // nwc, fw % nwc
        win = c * nfc * nwc + fw
        wb = win % 2
        pend_w[wb].wait()
        kpitch = lax.iota(jnp.int32, _LANES) * (chunk + 1)
        tvec = lr_v[0]

        def fbody(fi, _, wb=wb, wc=wc, fc=fc, kpitch=kpitch, tvec=tvec):
          fv = jnp.full((_LANES,), fi, dtype=jnp.int32)
          for wli in range(_WC):
            wv = jnp.full((_LANES,), wli, dtype=jnp.int32)
            vals = plsc.load_gather(wbufs[wb], [fv, wv, tvec])
            sidx = kpitch + ((wc * _WC + wli) * F + fc * _FC + fi)
            plsc.store_scatter(exb, [sidx], vals)
          return 0

        lax.fori_loop(0, _FC, fbody, 0)
        if win + 2 < nwin:
          issue_w(win + 2)

      for b1 in range(B):
        p = b1 * 2 + c
        if pend_o[0] is not None:
          pend_o[0].wait()

        def cbody(j, _, b1=b1):
          off = j * _LANES
          gl = exb[pl.ds(2 * b1 * (chunk + 1) + off, _LANES)]
          gr = exb[pl.ds((2 * b1 + 1) * (chunk + 1) + off, _LANES)]
          for b2 in range(B):
            ob[b2, pl.ds(off, _LANES)] = wr_v[b2] * gl + wl_v[b2] * gr
          return 0

        lax.fori_loop(0, nj, cbody, 0)
        pend_o[0] = pltpu.async_copy(
            ob, out_hbm.at[p, :, pl.ds(h * chunk, chunk)], osem)

    for cp in pend_o:
      if cp is not None:
        cp.wait()

  return sck(lr_b, wl_b, wr_b, gt0, gt1)


def kernel(patch_indices, grid0, grid1):
  T, F, H, W = grid0.shape
  B = patch_indices.shape[0]

  t = patch_indices[:, 0, 0, 0] * T
  left = jnp.floor(t).astype(jnp.int32)
  right = jnp.clip(left + 1, 0, T - 1)
  dl = t - left.astype(t.dtype)
  dr = right.astype(t.dtype) - t

  lr = jnp.stack([left, right], axis=1).reshape(-1)
  lr_b = lr[None, :]
  wl_b = jnp.broadcast_to(dl[:, None], (B, _LANES))
  wr_b = jnp.broadcast_to(dr[:, None], (B, _LANES))

  gt0 = grid0.transpose(1, 2, 3, 0)
  gt1 = grid1.transpose(1, 2, 3, 0)
  out = _sc_interp(lr_b, wl_b, wr_b, gt0, gt1, B, T, F, H, W)
  return out.reshape(B, 2, B, H, W, F).transpose(0, 1, 2, 5, 3, 4)


# speedup vs baseline: 2.2669x; 1.0342x over previous
import functools

import jax
import jax.numpy as jnp
from jax import lax
from jax.experimental import pallas as pl
from jax.experimental.pallas import tpu as pltpu
from jax.experimental.pallas import tpu_sc as plsc

_LANES = 16
_NW = 32
_FC = 16
_WC = 8
_TP = 256


def _sc_interp(lr_b, wl_b, wr_b, gt0, gt1, B, T, F, H, W):
  chunk = W * F
  npairs = 2 * B
  nfc = F // _FC
  nwc = W // _WC
  nj = chunk // _LANES
  mesh = plsc.VectorSubcoreMesh(core_axis_name="c", subcore_axis_name="s")

  @functools.partial(
      pl.kernel,
      mesh=mesh,
      compiler_params=pltpu.CompilerParams(needs_layout_passes=False),
      out_type=jax.ShapeDtypeStruct((npairs, B, H * chunk), jnp.float32),
      scratch_types=[
          pltpu.VMEM((1, _LANES), jnp.int32),
          pltpu.VMEM((B, _LANES), jnp.float32),
          pltpu.VMEM((B, _LANES), jnp.float32),
          pltpu.VMEM((_FC, _WC, T), jnp.float32),
          pltpu.VMEM((_FC, _WC, T), jnp.float32),
          pltpu.VMEM((npairs * (chunk + 1),), jnp.float32),
          pltpu.VMEM((B, chunk // 2), jnp.float32),
          pltpu.VMEM((B, chunk // 2), jnp.float32),
          pltpu.SemaphoreType.DMA,
          pltpu.SemaphoreType.DMA,
          pltpu.SemaphoreType.DMA,
          pltpu.SemaphoreType.DMA,
      ],
  )
  def sck(lr_hbm, wl_hbm, wr_hbm, g0_hbm, g1_hbm, out_hbm,
          lr_v, wl_v, wr_v, wbuf0, wbuf1, exb, ob0, ob1,
          wsem0, wsem1, osem0, osem1):
    h = lax.axis_index("s") * 2 + lax.axis_index("c")
    pltpu.sync_copy(lr_hbm, lr_v)
    pltpu.sync_copy(wl_hbm, wl_v)
    pltpu.sync_copy(wr_hbm, wr_v)

    wsems = [wsem0, wsem1]
    wbufs = [wbuf0, wbuf1]
    obufs = [ob0, ob1]
    osems = [osem0, osem1]
    pend_w = [None, None]
    pend_o = [None, None]
    nwin = 2 * nfc * nwc

    def issue_w(win):
      c, fw = win // (nfc * nwc), win % (nfc * nwc)
      fc, wc = fw // nwc, fw % nwc
      g = g0_hbm if c == 0 else g1_hbm
      wb = win % 2
      pend_w[wb] = pltpu.async_copy(
          g.at[pl.ds(fc * _FC, _FC), h, pl.ds(wc * _WC, _WC), :],
          wbufs[wb], wsems[wb])

    issue_w(0)
    issue_w(1)

    for c in range(2):
      for fw in range(nfc * nwc):
        fc, wc = fw // nwc, fw % nwc
        win = c * nfc * nwc + fw
        wb = win % 2
        pend_w[wb].wait()
        kpitch = lax.iota(jnp.int32, _LANES) * (chunk + 1)
        tvec = lr_v[0]

        def fbody(fi, _, wb=wb, wc=wc, fc=fc, kpitch=kpitch, tvec=tvec):
          fv = jnp.full((_LANES,), fi, dtype=jnp.int32)
          for wli in range(_WC):
            wv = jnp.full((_LANES,), wli, dtype=jnp.int32)
            vals = plsc.load_gather(wbufs[wb], [fv, wv, tvec])
            sidx = kpitch + ((wc * _WC + wli) * F + fc * _FC + fi)
            plsc.store_scatter(exb, [sidx], vals)
          return 0

        lax.fori_loop(0, _FC, fbody, 0)
        if win + 2 < nwin:
          issue_w(win + 2)

      half = chunk // 2
      for b1 in range(B):
        p = b1 * 2 + c
        for hf in range(2):
          s = (c * B + b1) * 2 + hf
          s %= 2
          ob = obufs[s]
          if pend_o[s] is not None:
            pend_o[s].wait()

          def cbody(j, _, b1=b1, ob=ob, hf=hf):
            off = j * _LANES
            src = hf * half + off
            gl = exb[pl.ds(2 * b1 * (chunk + 1) + src, _LANES)]
            gr = exb[pl.ds((2 * b1 + 1) * (chunk + 1) + src, _LANES)]
            for b2 in range(B):
              ob[b2, pl.ds(off, _LANES)] = wr_v[b2] * gl + wl_v[b2] * gr
            return 0

          lax.fori_loop(0, nj // 2, cbody, 0)
          pend_o[s] = pltpu.async_copy(
              ob, out_hbm.at[p, :, pl.ds(h * chunk + hf * half, half)],
              osems[s])

    for cp in pend_o:
      if cp is not None:
        cp.wait()

  return sck(lr_b, wl_b, wr_b, gt0, gt1)


def kernel(patch_indices, grid0, grid1):
  T, F, H, W = grid0.shape
  B = patch_indices.shape[0]

  t = patch_indices[:, 0, 0, 0] * T
  left = jnp.floor(t).astype(jnp.int32)
  right = jnp.clip(left + 1, 0, T - 1)
  dl = t - left.astype(t.dtype)
  dr = right.astype(t.dtype) - t

  lr = jnp.stack([left, right], axis=1).reshape(-1)
  lr_b = lr[None, :]
  wl_b = jnp.broadcast_to(dl[:, None], (B, _LANES))
  wr_b = jnp.broadcast_to(dr[:, None], (B, _LANES))

  gt0 = grid0.transpose(1, 2, 3, 0)
  gt1 = grid1.transpose(1, 2, 3, 0)
  out = _sc_interp(lr_b, wl_b, wr_b, gt0, gt1, B, T, F, H, W)
  return out.reshape(B, 2, B, H, W, F).transpose(0, 1, 2, 5, 3, 4)


# speedup vs baseline: 3.5217x; 1.5535x over previous
import functools

import jax
import jax.numpy as jnp
from jax import lax
from jax.experimental import pallas as pl
from jax.experimental.pallas import tpu as pltpu
from jax.experimental.pallas import tpu_sc as plsc

_LANES = 16
_NW = 32
_FC = 16
_WC = 8
_TP = 256


def _sc_interp(lr_b, wl_b, wr_b, gt0, gt1, B, T, F, H, W):
  chunk = W * F
  npairs = 2 * B
  nfc = F // _FC
  nwc = W // _WC
  nj = chunk // _LANES
  mesh = plsc.VectorSubcoreMesh(core_axis_name="c", subcore_axis_name="s")

  @functools.partial(
      pl.kernel,
      mesh=mesh,
      compiler_params=pltpu.CompilerParams(needs_layout_passes=False),
      out_type=jax.ShapeDtypeStruct((npairs, B, H * chunk), jnp.float32),
      scratch_types=[
          pltpu.VMEM((1, _LANES), jnp.int32),
          pltpu.VMEM((B, _LANES), jnp.float32),
          pltpu.VMEM((B, _LANES), jnp.float32),
          pltpu.VMEM((_FC, _WC, T), jnp.float32),
          pltpu.VMEM((_FC, _WC, T), jnp.float32),
          pltpu.VMEM((npairs * (chunk + 1),), jnp.float32),
          pltpu.VMEM((B, chunk // 2), jnp.float32),
          pltpu.VMEM((B, chunk // 2), jnp.float32),
          pltpu.SemaphoreType.DMA,
          pltpu.SemaphoreType.DMA,
          pltpu.SemaphoreType.DMA,
          pltpu.SemaphoreType.DMA,
      ],
  )
  def sck(lr_hbm, wl_hbm, wr_hbm, g0_hbm, g1_hbm, out_hbm,
          lr_v, wl_v, wr_v, wbuf0, wbuf1, exb, ob0, ob1,
          wsem0, wsem1, osem0, osem1):
    h = lax.axis_index("s") * 2 + lax.axis_index("c")
    pltpu.sync_copy(lr_hbm, lr_v)
    pltpu.sync_copy(wl_hbm, wl_v)
    pltpu.sync_copy(wr_hbm, wr_v)

    wsems = [wsem0, wsem1]
    wbufs = [wbuf0, wbuf1]
    obufs = [ob0, ob1]
    osems = [osem0, osem1]
    pend_w = [None, None]
    pend_o = [None, None]
    nwin = 2 * nfc * nwc

    def issue_w(win):
      c, fw = win // (nfc * nwc), win % (nfc * nwc)
      fc, wc = fw // nwc, fw % nwc
      g = g0_hbm if c == 0 else g1_hbm
      wb = win % 2
      pend_w[wb] = pltpu.async_copy(
          g.at[pl.ds(fc * _FC, _FC), h, pl.ds(wc * _WC, _WC), :],
          wbufs[wb], wsems[wb])

    issue_w(0)
    issue_w(1)

    for c in range(2):
      for fw in range(nfc * nwc):
        fc, wc = fw // nwc, fw % nwc
        win = c * nfc * nwc + fw
        wb = win % 2
        pend_w[wb].wait()
        kpitch = lax.iota(jnp.int32, _LANES) * (chunk + 1)
        tvec = lr_v[0]

        def fbody(fi, _, wb=wb, wc=wc, fc=fc, kpitch=kpitch, tvec=tvec):
          fv = jnp.full((_LANES,), fi, dtype=jnp.int32)
          for wli in range(_WC):
            wv = jnp.full((_LANES,), wli, dtype=jnp.int32)
            vals = plsc.load_gather(wbufs[wb], [fv, wv, tvec])
            sidx = kpitch + ((wc * _WC + wli) * F + fc * _FC + fi)
            plsc.store_scatter(exb, [sidx], vals)
          return 0

        lax.fori_loop(0, _FC, fbody, 0)
        if win + 2 < nwin:
          issue_w(win + 2)

      half = chunk // 2
      wls = [wl_v[b2] for b2 in range(B)]
      wrs = [wr_v[b2] for b2 in range(B)]
      for b1 in range(B):
        p = b1 * 2 + c
        for hf in range(2):
          s = (c * B + b1) * 2 + hf
          s %= 2
          ob = obufs[s]
          if pend_o[s] is not None:
            pend_o[s].wait()

          def cbody(j, _, b1=b1, ob=ob, hf=hf):
            off = j * _LANES
            src = hf * half + off
            gl = exb[pl.ds(2 * b1 * (chunk + 1) + src, _LANES)]
            gr = exb[pl.ds((2 * b1 + 1) * (chunk + 1) + src, _LANES)]
            for b2 in range(B):
              ob[b2, pl.ds(off, _LANES)] = wrs[b2] * gl + wls[b2] * gr
            return 0

          lax.fori_loop(0, nj // 2, cbody, 0)
          pend_o[s] = pltpu.async_copy(
              ob, out_hbm.at[p, :, pl.ds(h * chunk + hf * half, half)],
              osems[s])

    for cp in pend_o:
      if cp is not None:
        cp.wait()

  return sck(lr_b, wl_b, wr_b, gt0, gt1)


def kernel(patch_indices, grid0, grid1):
  T, F, H, W = grid0.shape
  B = patch_indices.shape[0]

  t = patch_indices[:, 0, 0, 0] * T
  left = jnp.floor(t).astype(jnp.int32)
  right = jnp.clip(left + 1, 0, T - 1)
  dl = t - left.astype(t.dtype)
  dr = right.astype(t.dtype) - t

  lr = jnp.stack([left, right], axis=1).reshape(-1)
  lr_b = lr[None, :]
  wl_b = jnp.broadcast_to(dl[:, None], (B, _LANES))
  wr_b = jnp.broadcast_to(dr[:, None], (B, _LANES))

  gt0 = grid0.transpose(1, 2, 3, 0)
  gt1 = grid1.transpose(1, 2, 3, 0)
  out = _sc_interp(lr_b, wl_b, wr_b, gt0, gt1, B, T, F, H, W)
  return out.reshape(B, 2, B, H, W, F).transpose(0, 1, 2, 5, 3, 4)
